# Initial kernel scaffold; baseline (speedup 1.0000x reference)
import functools

import jax
import jax.numpy as jnp
from jax.experimental import pallas as pl
from jax.experimental.pallas import tpu as pltpu

ALPHA = 1.702
LIMIT = 7.0
FP4_MAX = 6.0
FP8_MAX = 448.0


def _amax_body(x_ref, o_ref):
    j = pl.program_id(1)
    m = jnp.max(jnp.abs(x_ref[...]))

    @pl.when(j == 0)
    def _():
        o_ref[...] = jnp.zeros_like(o_ref)

    o_ref[...] = jnp.maximum(o_ref[...], m)


def _group_amax(x, bs):
    G, S, L = x.shape
    return pl.pallas_call(
        _amax_body,
        grid=(G, S // bs),
        in_specs=[pl.BlockSpec((1, bs, L), lambda g, j: (g, j, 0))],
        out_specs=pl.BlockSpec((1, 8, 128), lambda g, j: (g, 0, 0)),
        out_shape=jax.ShapeDtypeStruct((G, 8, 128), jnp.float32),
    )(x)


def _e4m3_roundtrip(v):
    return v.astype(jnp.float8_e4m3fn).astype(jnp.float32)


def _quant_body(x_ref, gm_ref, o_ref, *, bs):
    x = x_ref[0]
    L = x.shape[1]
    gamax = jnp.maximum(jnp.max(gm_ref[...]), 1e-12)
    gscale = FP8_MAX * FP4_MAX / gamax
    xr = x.reshape(bs // 16, 16, L)
    amax = jnp.max(jnp.abs(xr), axis=1, keepdims=True)
    sf = _e4m3_roundtrip(amax / FP4_MAX * gscale)
    sf_safe = jnp.where(sf <= 0.0, 1.0, sf)
    scaled = xr * gscale / sf_safe
    sign = jnp.sign(scaled)
    a = jnp.clip(jnp.abs(scaled), 0.0, FP4_MAX)
    q = jnp.where(a < 0.25, 0.0,
        jnp.where(a < 0.75, 0.5,
        jnp.where(a < 1.25, 1.0,
        jnp.where(a < 1.75, 1.5,
        jnp.where(a < 2.5, 2.0,
        jnp.where(a < 3.5, 3.0,
        jnp.where(a < 5.0, 4.0, FP4_MAX)))))))
    deq = q * sign * sf_safe / gscale
    deq = jnp.where(amax <= 0.0, 0.0, deq)
    o_ref[0] = deq.reshape(bs, L).astype(jnp.bfloat16)


def _quant(x, gm, bs):
    G, S, L = x.shape
    return pl.pallas_call(
        functools.partial(_quant_body, bs=bs),
        grid=(G, S // bs),
        in_specs=[
            pl.BlockSpec((1, bs, L), lambda g, j: (g, j, 0)),
            pl.BlockSpec((1, 8, 128), lambda g, j: (g, 0, 0)),
        ],
        out_specs=pl.BlockSpec((1, bs, L), lambda g, j: (g, j, 0)),
        out_shape=jax.ShapeDtypeStruct((G, S, L), jnp.bfloat16),
    )(x, gm)


def _moe_body(xq_ref, gq_ref, uq_ref, dq_ref, bg_ref, bu_ref, bd_ref,
              ri_ref, rw_ref, o_ref, acc_ref, *, NE, NK, BT, TOPK):
    pe = pl.program_id(0)
    pk = pl.program_id(1)
    pi = pl.program_id(2)

    x = xq_ref[...]
    g = jnp.dot(x, gq_ref[0], preferred_element_type=jnp.float32) + bg_ref[0]
    u = jnp.dot(x, uq_ref[0], preferred_element_type=jnp.float32) + bu_ref[0]
    g = jnp.minimum(g, LIMIT)
    u = jnp.clip(u, -LIMIT, LIMIT)
    act = ((u + 1.0) * (g * jax.nn.sigmoid(g * ALPHA))).astype(jnp.bfloat16)
    o = jnp.dot(act, dq_ref[0], preferred_element_type=jnp.float32)
    o = jnp.where(pk == 0, o + bd_ref[0], o)

    ri = ri_ref[...]
    rw = rw_ref[...]
    eio = jax.lax.broadcasted_iota(jnp.int32, rw.shape, 1)
    c = jnp.zeros((x.shape[0], 1), jnp.float32)
    for t in range(TOPK):
        rk = ri[:, t:t + 1]
        vk = rk < NE
        sk = jnp.clip(rk, 0, NE - 1)
        wk = jnp.sum(rw * (sk == eio).astype(jnp.float32), axis=1,
                     keepdims=True)
        c = c + jnp.where((sk == pe) & vk, wk, 0.0)

    contrib = c * o
    row = pi * BT

    @pl.when((pe == 0) & (pk == 0))
    def _():
        acc_ref[pl.ds(row, BT), :] = contrib

    @pl.when((pe > 0) | (pk > 0))
    def _():
        acc_ref[pl.ds(row, BT), :] = acc_ref[pl.ds(row, BT), :] + contrib

    o_ref[...] = acc_ref[pl.ds(row, BT), :]


def _moe_call(xq, gq, uq, dq, bg, bu, bd, ri, rw, BT=256, BI=512):
    T, Hh = xq.shape
    NE, _, Ii = gq.shape
    NK = Ii // BI
    NT = T // BT
    TOPK = ri.shape[1]
    return pl.pallas_call(
        functools.partial(_moe_body, NE=NE, NK=NK, BT=BT, TOPK=TOPK),
        grid=(NE, NK, NT),
        in_specs=[
            pl.BlockSpec((BT, Hh), lambda e, k, i: (i, 0)),
            pl.BlockSpec((1, Hh, BI), lambda e, k, i: (e, 0, k)),
            pl.BlockSpec((1, Hh, BI), lambda e, k, i: (e, 0, k)),
            pl.BlockSpec((1, BI, Hh), lambda e, k, i: (e, k, 0)),
            pl.BlockSpec((1, 1, BI), lambda e, k, i: (e, 0, k)),
            pl.BlockSpec((1, 1, BI), lambda e, k, i: (e, 0, k)),
            pl.BlockSpec((1, 1, Hh), lambda e, k, i: (e, 0, 0)),
            pl.BlockSpec((BT, ri.shape[1]), lambda e, k, i: (i, 0)),
            pl.BlockSpec((BT, rw.shape[1]), lambda e, k, i: (i, 0)),
        ],
        out_specs=pl.BlockSpec((BT, Hh), lambda e, k, i: (i, 0)),
        out_shape=jax.ShapeDtypeStruct((T, Hh), jnp.float32),
        scratch_shapes=[pltpu.VMEM((T, Hh), jnp.float32)],
    )(xq, gq, uq, dq, bg, bu, bd, ri, rw)


def kernel(hidden_states, gate_up_proj, gate_up_proj_bias, down_proj,
           down_proj_bias, router_indices, routing_weights):
    T, Hh = hidden_states.shape
    NE, _, I2 = gate_up_proj.shape
    Ii = I2 // 2

    gm_w13 = _group_amax(gate_up_proj, bs=256)
    gm_w2 = _group_amax(down_proj, bs=256)
    xT = hidden_states.T
    gm_x = _group_amax(xT[None], bs=256)

    gateW = gate_up_proj[:, :, 0::2]
    upW = gate_up_proj[:, :, 1::2]
    gq = _quant(gateW, gm_w13, bs=256)
    uq = _quant(upW, gm_w13, bs=256)
    dq = _quant(down_proj, gm_w2, bs=256)
    xq = _quant(xT[None], gm_x, bs=256)[0].T

    bg = gate_up_proj_bias[:, 0::2].reshape(NE, 1, Ii)
    bu = gate_up_proj_bias[:, 1::2].reshape(NE, 1, Ii)
    bd = down_proj_bias.reshape(NE, 1, Hh)

    return _moe_call(xq, gq, uq, dq, bg, bu, bd, router_indices,
                     routing_weights)


# speedup vs baseline: 382.9084x; 382.9084x over previous
import functools

import jax
import jax.numpy as jnp
from jax.experimental import pallas as pl
from jax.experimental.pallas import tpu as pltpu

ALPHA = 1.702
LIMIT = 7.0
FP4_MAX = 6.0
FP8_MAX = 448.0


def _amax_body(x_ref, o_ref):
    j = pl.program_id(1)
    m = jnp.max(jnp.abs(x_ref[...]))

    @pl.when(j == 0)
    def _():
        o_ref[...] = jnp.zeros_like(o_ref)

    o_ref[...] = jnp.maximum(o_ref[...], m)


def _group_amax(x, bs):
    G, S, L = x.shape
    return pl.pallas_call(
        _amax_body,
        grid=(G, S // bs),
        in_specs=[pl.BlockSpec((1, bs, L), lambda g, j: (g, j, 0))],
        out_specs=pl.BlockSpec((1, 8, 128), lambda g, j: (g, 0, 0)),
        out_shape=jax.ShapeDtypeStruct((G, 8, 128), jnp.float32),
    )(x)


def _e4m3_roundtrip(v):
    bits = jax.lax.bitcast_convert_type(v, jnp.int32)
    e = (bits >> 23) & 0xFF
    e_eff = jnp.maximum(e - 127, -6)
    ulp = jax.lax.bitcast_convert_type((e_eff + 124) << 23, jnp.float32)
    q = jnp.rint(v / ulp) * ulp
    return jnp.minimum(q, FP8_MAX)


def _quant_body(x_ref, gm_ref, o_ref, *, bs, rtne_deq):
    x = x_ref[0]
    L = x.shape[1]
    gamax = jnp.maximum(jnp.max(gm_ref[...]), 1e-12)
    gscale = FP8_MAX * FP4_MAX / gamax
    xr = x.reshape(bs // 16, 16, L)
    amax = jnp.max(jnp.abs(xr), axis=1, keepdims=True)
    pre = amax / FP4_MAX * gscale
    sf_deq = _e4m3_roundtrip(pre) if rtne_deq else pre
    sfb = jnp.where(pre <= 0.0, 1.0, pre)
    sfd = jnp.where(sf_deq <= 0.0, 1.0, sf_deq)
    scaled = xr * gscale / sfb
    sign = jnp.sign(scaled)
    a = jnp.clip(jnp.abs(scaled), 0.0, FP4_MAX)
    q = jnp.where(a < 0.25, 0.0,
        jnp.where(a < 0.75, 0.5,
        jnp.where(a < 1.25, 1.0,
        jnp.where(a < 1.75, 1.5,
        jnp.where(a < 2.5, 2.0,
        jnp.where(a < 3.5, 3.0,
        jnp.where(a < 5.0, 4.0, FP4_MAX)))))))
    deq = q * sign * sfd / gscale
    deq = jnp.where(amax <= 0.0, 0.0, deq)
    o_ref[0] = deq.reshape(bs, L).astype(jnp.bfloat16)


def _quant(x, gm, bs, rtne_deq=False):
    G, S, L = x.shape
    return pl.pallas_call(
        functools.partial(_quant_body, bs=bs, rtne_deq=rtne_deq),
        grid=(G, S // bs),
        in_specs=[
            pl.BlockSpec((1, bs, L), lambda g, j: (g, j, 0)),
            pl.BlockSpec((1, 8, 128), lambda g, j: (g, 0, 0)),
        ],
        out_specs=pl.BlockSpec((1, bs, L), lambda g, j: (g, j, 0)),
        out_shape=jax.ShapeDtypeStruct((G, S, L), jnp.bfloat16),
    )(x, gm)


def _moe_body(xq_ref, gq_ref, uq_ref, dq_ref, bg_ref, bu_ref, bd_ref,
              ri_ref, rw_ref, o_ref, acc_ref, *, NE, NK, BT, TOPK):
    pe = pl.program_id(0)
    pk = pl.program_id(1)
    pi = pl.program_id(2)

    x = xq_ref[...]
    g = jnp.dot(x, gq_ref[0], preferred_element_type=jnp.float32) + bg_ref[0]
    u = jnp.dot(x, uq_ref[0], preferred_element_type=jnp.float32) + bu_ref[0]
    g = jnp.minimum(g, LIMIT)
    u = jnp.clip(u, -LIMIT, LIMIT)
    act = ((u + 1.0) * (g * jax.nn.sigmoid(g * ALPHA))).astype(jnp.bfloat16)
    o = jnp.dot(act, dq_ref[0], preferred_element_type=jnp.float32)
    o = jnp.where(pk == 0, o + bd_ref[0], o)

    ri = ri_ref[...]
    rw = rw_ref[...]
    eio = jax.lax.broadcasted_iota(jnp.int32, rw.shape, 1)
    c = jnp.zeros((x.shape[0], 1), jnp.float32)
    for t in range(TOPK):
        rk = ri[:, t:t + 1]
        vk = rk < NE
        sk = jnp.clip(rk, 0, NE - 1)
        wk = jnp.sum(rw * (sk == eio).astype(jnp.float32), axis=1,
                     keepdims=True)
        c = c + jnp.where((sk == pe) & vk, wk, 0.0)

    contrib = c * o
    row = pi * BT

    @pl.when((pe == 0) & (pk == 0))
    def _():
        acc_ref[pl.ds(row, BT), :] = contrib

    @pl.when((pe > 0) | (pk > 0))
    def _():
        acc_ref[pl.ds(row, BT), :] = acc_ref[pl.ds(row, BT), :] + contrib

    o_ref[...] = acc_ref[pl.ds(row, BT), :]


def _moe_call(xq, gq, uq, dq, bg, bu, bd, ri, rw, BT=256, BI=512):
    T, Hh = xq.shape
    NE, _, Ii = gq.shape
    NK = Ii // BI
    NT = T // BT
    TOPK = ri.shape[1]
    return pl.pallas_call(
        functools.partial(_moe_body, NE=NE, NK=NK, BT=BT, TOPK=TOPK),
        grid=(NE, NK, NT),
        in_specs=[
            pl.BlockSpec((BT, Hh), lambda e, k, i: (i, 0)),
            pl.BlockSpec((1, Hh, BI), lambda e, k, i: (e, 0, k)),
            pl.BlockSpec((1, Hh, BI), lambda e, k, i: (e, 0, k)),
            pl.BlockSpec((1, BI, Hh), lambda e, k, i: (e, k, 0)),
            pl.BlockSpec((1, 1, BI), lambda e, k, i: (e, 0, k)),
            pl.BlockSpec((1, 1, BI), lambda e, k, i: (e, 0, k)),
            pl.BlockSpec((1, 1, Hh), lambda e, k, i: (e, 0, 0)),
            pl.BlockSpec((BT, ri.shape[1]), lambda e, k, i: (i, 0)),
            pl.BlockSpec((BT, rw.shape[1]), lambda e, k, i: (i, 0)),
        ],
        out_specs=pl.BlockSpec((BT, Hh), lambda e, k, i: (i, 0)),
        out_shape=jax.ShapeDtypeStruct((T, Hh), jnp.float32),
        scratch_shapes=[pltpu.VMEM((T, Hh), jnp.float32)],
    )(xq, gq, uq, dq, bg, bu, bd, ri, rw)


def kernel(hidden_states, gate_up_proj, gate_up_proj_bias, down_proj,
           down_proj_bias, router_indices, routing_weights):
    T, Hh = hidden_states.shape
    NE, _, I2 = gate_up_proj.shape
    Ii = I2 // 2

    gm_w13 = _group_amax(gate_up_proj, bs=256)
    gm_w2 = _group_amax(down_proj, bs=256)
    xT = hidden_states.T
    gm_x = _group_amax(xT[None], bs=256)

    gateW = gate_up_proj[:, :, 0::2]
    upW = gate_up_proj[:, :, 1::2]
    gq = _quant(gateW, gm_w13, bs=256)
    uq = _quant(upW, gm_w13, bs=256)
    dq = _quant(down_proj, gm_w2, bs=256)
    xq = _quant(xT[None], gm_x, bs=256, rtne_deq=True)[0].T

    bg = gate_up_proj_bias[:, 0::2].reshape(NE, 1, Ii)
    bu = gate_up_proj_bias[:, 1::2].reshape(NE, 1, Ii)
    bd = down_proj_bias.reshape(NE, 1, Hh)

    return _moe_call(xq, gq, uq, dq, bg, bu, bd, router_indices,
                     routing_weights)
